# baseline (speedup 1.0000x reference)
import functools

import jax
import jax.numpy as jnp
from jax import lax
from jax.experimental import pallas as pl
from jax.experimental.pallas import tpu as pltpu
from jax.experimental.pallas import tpu_sc as plsc

T = 4096
D_MODEL = 1024
D_FF = 4096
E = 8
TOPK = 2
R = 64

TILE_T = 256

SC_NC = 2
SC_NS = 16
SC_LANES = 16
SC_WORKERS = SC_NC * SC_NS
TOK_PER_WORKER = T // SC_WORKERS
CHUNKS = TOK_PER_WORKER // SC_LANES


PREP_GRID = 4
PREP_TILE = T // PREP_GRID


def _prep_kernel(x_ref, wr_ref, a_ref, out_ref, a2d_ref):
    l = jnp.dot(x_ref[...], wr_ref[...], preferred_element_type=jnp.float32)
    out_ref[...] = l.T
    a2d_ref[:, :R] = a_ref[0]
    a2d_ref[:, R:] = a_ref[1]


def _router_sc_kernel(logits_hbm, w_hbm, l_v, w_v):
    wid = lax.axis_index("s") * SC_NC + lax.axis_index("c")
    base = wid * TOK_PER_WORKER
    pltpu.sync_copy(logits_hbm.at[:, pl.ds(base, TOK_PER_WORKER)], l_v)
    for c in range(CHUNKS):
        sl = pl.ds(c * SC_LANES, SC_LANES)
        l = [l_v[e, sl] for e in range(E)]
        m = functools.reduce(jnp.maximum, l)
        ee = [jnp.exp(le - m) for le in l]
        z = functools.reduce(lambda a, b: a + b, ee)
        v1 = l[0]
        p1 = ee[0]
        i1 = jnp.zeros((SC_LANES,), jnp.int32)
        for e in range(1, E):
            gt = l[e] > v1
            v1 = jnp.where(gt, l[e], v1)
            p1 = jnp.where(gt, ee[e], p1)
            i1 = jnp.where(gt, e, i1)
        v2 = jnp.full((SC_LANES,), -jnp.inf, jnp.float32)
        p2 = jnp.zeros((SC_LANES,), jnp.float32)
        i2 = jnp.full((SC_LANES,), E, jnp.int32)
        for e in range(E):
            gt = (l[e] > v2) & (i1 != e)
            v2 = jnp.where(gt, l[e], v2)
            p2 = jnp.where(gt, ee[e], p2)
            i2 = jnp.where(gt, e, i2)
        q1 = p1 / z
        q2 = p2 / z
        den = q1 + q2 + 1e-9
        g1 = q1 / den
        g2 = q2 / den
        for e in range(E):
            w_v[e, sl] = jnp.where(i1 == e, g1, 0.0) + jnp.where(i2 == e, g2, 0.0)
    pltpu.sync_copy(w_v, w_hbm.at[:, pl.ds(base, TOK_PER_WORKER)])


_router_sc = functools.partial(
    pl.kernel,
    mesh=plsc.VectorSubcoreMesh(core_axis_name="c", subcore_axis_name="s"),
    out_type=jax.ShapeDtypeStruct((E, T), jnp.float32),
    scratch_types=[
        pltpu.VMEM((E, TOK_PER_WORKER), jnp.float32),
        pltpu.VMEM((E, TOK_PER_WORKER), jnp.float32),
    ],
)(_router_sc_kernel)


def _moe_kernel(x_ref, w1_ref, b1_ref, w2_ref, b2_ref, wt_ref, a_ref,
                b_lora_ref, scale_ref, out_ref):
    x = x_ref[...]
    h = jax.nn.gelu(
        jnp.dot(x, w1_ref[...], preferred_element_type=jnp.float32) + b1_ref[...]
    )
    base = jnp.dot(h, w2_ref[...], preferred_element_type=jnp.float32) + b2_ref[...]

    wscale = wt_ref[...].T * scale_ref[...]

    low = jnp.dot(h, a_ref[...], preferred_element_type=jnp.float32)
    col_e = jax.lax.broadcasted_iota(jnp.int32, low.shape, 1) // R
    gm = jnp.zeros_like(low)
    for e in range(E):
        gm = jnp.where(col_e == e, wscale[:, e:e + 1], gm)
    moe = jnp.dot(low * gm, b_lora_ref[...], preferred_element_type=jnp.float32)

    out_ref[...] = base + moe


@jax.jit
def kernel(x, W1, b1, W2, b2, Wr, A, B, scale):
    b2d = B.reshape(E * R, D_MODEL)

    logits_t, a2d = pl.pallas_call(
        _prep_kernel,
        grid=(PREP_GRID,),
        in_specs=[
            pl.BlockSpec((PREP_TILE, D_MODEL), lambda i: (i, 0)),
            pl.BlockSpec((D_MODEL, E), lambda i: (0, 0)),
            pl.BlockSpec((2, D_FF, R), lambda i: (i, 0, 0)),
        ],
        out_specs=[
            pl.BlockSpec((E, PREP_TILE), lambda i: (0, i)),
            pl.BlockSpec((D_FF, 2 * R), lambda i: (0, i)),
        ],
        out_shape=[
            jax.ShapeDtypeStruct((E, T), jnp.float32),
            jax.ShapeDtypeStruct((D_FF, E * R), jnp.float32),
        ],
    )(x, Wr, A)

    w_t = _router_sc(logits_t)

    grid = (T // TILE_T,)
    full = lambda i: (0, 0)
    out = pl.pallas_call(
        _moe_kernel,
        grid=grid,
        in_specs=[
            pl.BlockSpec((TILE_T, D_MODEL), lambda i: (i, 0)),
            pl.BlockSpec((D_MODEL, D_FF), full),
            pl.BlockSpec((1, D_FF), full),
            pl.BlockSpec((D_FF, D_MODEL), full),
            pl.BlockSpec((1, D_MODEL), full),
            pl.BlockSpec((E, TILE_T), lambda i: (0, i)),
            pl.BlockSpec((D_FF, E * R), full),
            pl.BlockSpec((E * R, D_MODEL), full),
            pl.BlockSpec((1, E), full),
        ],
        out_specs=pl.BlockSpec((TILE_T, D_MODEL), lambda i: (i, 0)),
        out_shape=jax.ShapeDtypeStruct((T, D_MODEL), jnp.float32),
    )(x, W1, b1.reshape(1, D_FF), W2, b2.reshape(1, D_MODEL), w_t,
      a2d, b2d, scale.reshape(1, E))
    return out


# speedup vs baseline: 1.1621x; 1.1621x over previous
import functools

import jax
import jax.numpy as jnp
from jax import lax
from jax.experimental import pallas as pl
from jax.experimental.pallas import tpu as pltpu
from jax.experimental.pallas import tpu_sc as plsc

T = 4096
D_MODEL = 1024
D_FF = 4096
E = 8
TOPK = 2
R = 64

TILE_T = 512

SC_NC = 2
SC_NS = 16
SC_LANES = 16
SC_WORKERS = SC_NC * SC_NS
TOK_PER_WORKER = T // SC_WORKERS
CHUNKS = TOK_PER_WORKER // SC_LANES


PREP_GRID = 4
PREP_TILE = T // PREP_GRID


def _logits_kernel(x_ref, wr_ref, out_ref):
    l = jnp.dot(x_ref[...], wr_ref[...], preferred_element_type=jnp.float32)
    out_ref[...] = l.T


def _router_sc_kernel(logits_hbm, w_hbm, l_v, w_v):
    wid = lax.axis_index("s") * SC_NC + lax.axis_index("c")
    base = wid * TOK_PER_WORKER
    pltpu.sync_copy(logits_hbm.at[:, pl.ds(base, TOK_PER_WORKER)], l_v)
    for c in range(CHUNKS):
        sl = pl.ds(c * SC_LANES, SC_LANES)
        l = [l_v[e, sl] for e in range(E)]
        m = functools.reduce(jnp.maximum, l)
        ee = [jnp.exp(le - m) for le in l]
        z = functools.reduce(lambda a, b: a + b, ee)
        v1 = l[0]
        p1 = ee[0]
        i1 = jnp.zeros((SC_LANES,), jnp.int32)
        for e in range(1, E):
            gt = l[e] > v1
            v1 = jnp.where(gt, l[e], v1)
            p1 = jnp.where(gt, ee[e], p1)
            i1 = jnp.where(gt, e, i1)
        v2 = jnp.full((SC_LANES,), -jnp.inf, jnp.float32)
        p2 = jnp.zeros((SC_LANES,), jnp.float32)
        i2 = jnp.full((SC_LANES,), E, jnp.int32)
        for e in range(E):
            gt = (l[e] > v2) & (i1 != e)
            v2 = jnp.where(gt, l[e], v2)
            p2 = jnp.where(gt, ee[e], p2)
            i2 = jnp.where(gt, e, i2)
        q1 = p1 / z
        q2 = p2 / z
        den = q1 + q2 + 1e-9
        g1 = q1 / den
        g2 = q2 / den
        for e in range(E):
            w_v[e, sl] = jnp.where(i1 == e, g1, 0.0) + jnp.where(i2 == e, g2, 0.0)
    pltpu.sync_copy(w_v, w_hbm.at[:, pl.ds(base, TOK_PER_WORKER)])


_router_sc = functools.partial(
    pl.kernel,
    mesh=plsc.VectorSubcoreMesh(core_axis_name="c", subcore_axis_name="s"),
    out_type=jax.ShapeDtypeStruct((E, T), jnp.float32),
    scratch_types=[
        pltpu.VMEM((E, TOK_PER_WORKER), jnp.float32),
        pltpu.VMEM((E, TOK_PER_WORKER), jnp.float32),
    ],
)(_router_sc_kernel)


def _moe_kernel(x_ref, w1_ref, b1_ref, w2_ref, b2_ref, wt_ref, a_ref,
                b_lora_ref, scale_ref, out_ref):
    x = x_ref[...]
    h = jax.nn.gelu(
        jnp.dot(x, w1_ref[...], preferred_element_type=jnp.float32) + b1_ref[...]
    )
    base = jnp.dot(h, w2_ref[...], preferred_element_type=jnp.float32) + b2_ref[...]

    wscale = wt_ref[...].T * scale_ref[...]

    low = jnp.dot(h, a_ref[...], preferred_element_type=jnp.float32)
    col_e = jax.lax.broadcasted_iota(jnp.int32, low.shape, 1) // R
    gm = jnp.zeros_like(low)
    for e in range(E):
        gm = jnp.where(col_e == e, wscale[:, e:e + 1], gm)
    moe = jnp.dot(low * gm, b_lora_ref[...], preferred_element_type=jnp.float32)

    out_ref[...] = base + moe


@jax.jit
def kernel(x, W1, b1, W2, b2, Wr, A, B, scale):
    b2d = B.reshape(E * R, D_MODEL)

    logits_t = pl.pallas_call(
        _logits_kernel,
        grid=(PREP_GRID,),
        in_specs=[
            pl.BlockSpec((PREP_TILE, D_MODEL), lambda i: (i, 0)),
            pl.BlockSpec((D_MODEL, E), lambda i: (0, 0)),
        ],
        out_specs=pl.BlockSpec((E, PREP_TILE), lambda i: (0, i)),
        out_shape=jax.ShapeDtypeStruct((E, T), jnp.float32),
    )(x, Wr)

    a_tied, logits_t = jax.lax.optimization_barrier((A, logits_t))
    a2d = jnp.transpose(a_tied, (1, 0, 2)).reshape(D_FF, E * R)

    w_t = _router_sc(logits_t)

    grid = (T // TILE_T,)
    full = lambda i: (0, 0)
    out = pl.pallas_call(
        _moe_kernel,
        grid=grid,
        in_specs=[
            pl.BlockSpec((TILE_T, D_MODEL), lambda i: (i, 0)),
            pl.BlockSpec((D_MODEL, D_FF), full),
            pl.BlockSpec((1, D_FF), full),
            pl.BlockSpec((D_FF, D_MODEL), full),
            pl.BlockSpec((1, D_MODEL), full),
            pl.BlockSpec((E, TILE_T), lambda i: (0, i)),
            pl.BlockSpec((D_FF, E * R), full),
            pl.BlockSpec((E * R, D_MODEL), full),
            pl.BlockSpec((1, E), full),
        ],
        out_specs=pl.BlockSpec((TILE_T, D_MODEL), lambda i: (i, 0)),
        out_shape=jax.ShapeDtypeStruct((T, D_MODEL), jnp.float32),
        compiler_params=pltpu.CompilerParams(
            vmem_limit_bytes=100 * 1024 * 1024,
        ),
    )(x, W1, b1.reshape(1, D_FF), W2, b2.reshape(1, D_MODEL), w_t,
      a2d, b2d, scale.reshape(1, E))
    return out
